# initial kernel scaffold (unmeasured)
import jax
import jax.numpy as jnp
from jax import lax
from jax.experimental import pallas as pl
from jax.experimental.pallas import tpu as pltpu


def kernel(
    x,
):
    def body(*refs):
        pass

    out_shape = jax.ShapeDtypeStruct(..., jnp.float32)
    return pl.pallas_call(body, out_shape=out_shape)(...)


# baseline (device time: 31985 ns/iter reference)
import jax
import jax.numpy as jnp
from jax import lax
from jax.experimental import pallas as pl
from jax.experimental.pallas import tpu as pltpu

HALF = 512


def kernel(x):
    m_per, n = x.shape

    def body(x_ref, out_ref, send_sems, recv_sems):
        my_x = lax.axis_index("x")
        my_y = lax.axis_index("y")
        other_x = 1 - my_x
        other_y = 1 - my_y

        barrier_sem = pltpu.get_barrier_semaphore()
        pl.semaphore_signal(
            barrier_sem, inc=1,
            device_id=(other_x, my_y), device_id_type=pl.DeviceIdType.MESH,
        )
        pl.semaphore_signal(
            barrier_sem, inc=1,
            device_id=(my_x, other_y), device_id_type=pl.DeviceIdType.MESH,
        )
        pl.semaphore_wait(barrier_sem, 2)

        src_off = my_y * HALF
        dst_off = my_x * m_per + my_y * HALF
        rdma_x = pltpu.make_async_remote_copy(
            src_ref=x_ref.at[pl.ds(src_off, HALF)],
            dst_ref=out_ref.at[pl.ds(dst_off, HALF)],
            send_sem=send_sems.at[0],
            recv_sem=recv_sems.at[0],
            device_id=(other_x, my_y),
            device_id_type=pl.DeviceIdType.MESH,
        )
        rdma_x.start()

        out_ref[pl.ds(my_x * m_per, m_per), :] = x_ref[:, :]

        rdma_x.wait()

        fwd_off = other_x * m_per + my_y * HALF
        rdma_y = pltpu.make_async_remote_copy(
            src_ref=out_ref.at[pl.ds(fwd_off, HALF)],
            dst_ref=out_ref.at[pl.ds(fwd_off, HALF)],
            send_sem=send_sems.at[1],
            recv_sem=recv_sems.at[1],
            device_id=(my_x, other_y),
            device_id_type=pl.DeviceIdType.MESH,
        )
        rdma_y.start()
        rdma_y.wait()

    return pl.pallas_call(
        body,
        out_shape=jax.ShapeDtypeStruct((2 * m_per, n), x.dtype),
        in_specs=[pl.BlockSpec(memory_space=pltpu.VMEM)],
        out_specs=pl.BlockSpec(memory_space=pltpu.VMEM),
        scratch_shapes=[
            pltpu.SemaphoreType.DMA((2,)),
            pltpu.SemaphoreType.DMA((2,)),
        ],
        compiler_params=pltpu.CompilerParams(collective_id=0),
    )(x)


# device time: 22501 ns/iter; 1.4215x vs baseline; 1.4215x over previous
import jax
import jax.numpy as jnp
from jax import lax
from jax.experimental import pallas as pl
from jax.experimental.pallas import tpu as pltpu

HALF = 512
K = 8
CH = HALF // K


def kernel(x):
    m_per, n = x.shape

    def body(x_ref, out_ref, sx_sems, rx_sems, sy_sems, ry_sems):
        my_x = lax.axis_index("x")
        my_y = lax.axis_index("y")
        other_x = 1 - my_x
        other_y = 1 - my_y

        barrier_sem = pltpu.get_barrier_semaphore()
        pl.semaphore_signal(
            barrier_sem, inc=1,
            device_id=(other_x, my_y), device_id_type=pl.DeviceIdType.MESH,
        )
        pl.semaphore_signal(
            barrier_sem, inc=1,
            device_id=(my_x, other_y), device_id_type=pl.DeviceIdType.MESH,
        )
        pl.semaphore_wait(barrier_sem, 2)

        src_off = my_y * HALF
        dst_off = my_x * m_per + my_y * HALF
        rdmas_x = []
        for c in range(K):
            r = pltpu.make_async_remote_copy(
                src_ref=x_ref.at[pl.ds(src_off + c * CH, CH)],
                dst_ref=out_ref.at[pl.ds(dst_off + c * CH, CH)],
                send_sem=sx_sems.at[c],
                recv_sem=rx_sems.at[c],
                device_id=(other_x, my_y),
                device_id_type=pl.DeviceIdType.MESH,
            )
            r.start()
            rdmas_x.append(r)

        out_ref[pl.ds(my_x * m_per, m_per), :] = x_ref[:, :]

        fwd_off = other_x * m_per + my_y * HALF
        rdmas_y = []
        for c in range(K):
            rdmas_x[c].wait_recv()
            r = pltpu.make_async_remote_copy(
                src_ref=out_ref.at[pl.ds(fwd_off + c * CH, CH)],
                dst_ref=out_ref.at[pl.ds(fwd_off + c * CH, CH)],
                send_sem=sy_sems.at[c],
                recv_sem=ry_sems.at[c],
                device_id=(my_x, other_y),
                device_id_type=pl.DeviceIdType.MESH,
            )
            r.start()
            rdmas_y.append(r)

        for c in range(K):
            rdmas_y[c].wait_recv()
        for c in range(K):
            rdmas_x[c].wait_send()
            rdmas_y[c].wait_send()

    return pl.pallas_call(
        body,
        out_shape=jax.ShapeDtypeStruct((2 * m_per, n), x.dtype),
        in_specs=[pl.BlockSpec(memory_space=pltpu.VMEM)],
        out_specs=pl.BlockSpec(memory_space=pltpu.VMEM),
        scratch_shapes=[
            pltpu.SemaphoreType.DMA((K,)),
            pltpu.SemaphoreType.DMA((K,)),
            pltpu.SemaphoreType.DMA((K,)),
            pltpu.SemaphoreType.DMA((K,)),
        ],
        compiler_params=pltpu.CompilerParams(collective_id=0),
    )(x)
